# baseline (speedup 1.0000x reference)
import functools

import jax
import jax.numpy as jnp
from jax import lax
from jax.experimental import pallas as pl
from jax.experimental.pallas import tpu as pltpu
from jax.experimental.pallas import tpu_sc as plsc

DIM = 64
NC = 2
NS = 16
NW = NC * NS
LANES = 16

CH = 128
NBUF = 10
SUB = CH // LANES


def _sc_gather(idx, kern, priori):
  B = idx.shape[0]
  rows_w = B // NW
  nch = rows_w // CH
  groups = nch // NBUF

  mesh = plsc.VectorSubcoreMesh(
      core_axis_name="c", subcore_axis_name="s",
      num_cores=NC, num_subcores=NS)

  @functools.partial(
      pl.kernel,
      out_type=jax.ShapeDtypeStruct((B, 2 * DIM), jnp.float32),
      mesh=mesh,
      compiler_params=pltpu.CompilerParams(use_tc_tiling_on_sc=False),
      scratch_types=[
          pltpu.VMEM((rows_w,), jnp.int32),
          pltpu.VMEM((nch, CH), jnp.int32),
          pltpu.VMEM((2, DIM), jnp.float32),
          pltpu.VMEM((nch * LANES,), jnp.int32),
          pltpu.VMEM((2 * LANES,), jnp.int32),
          pltpu.VMEM((NBUF, CH, DIM), jnp.float32),
          pltpu.SemaphoreType.DMA((NBUF,)),
          pltpu.SemaphoreType.DMA((NBUF,)),
      ],
  )
  def body(idx_hbm, kern_hbm, priori_hbm, out_hbm,
           idx_v, aidx_v, kern_v, flags_v, fold_v, bufs, gsem, osem):
    wid = lax.axis_index("s") * NC + lax.axis_index("c")
    base = wid * rows_w

    pltpu.sync_copy(idx_hbm.at[pl.ds(base, rows_w)], idx_v)
    pltpu.sync_copy(kern_hbm, kern_v)

    def prep(c, carry):
      mm = None
      for s in range(SUB):
        v = idx_v[pl.ds(c * CH + s * LANES, LANES)]
        aidx_v[c, pl.ds(s * LANES, LANES)] = jnp.maximum(v - 2, 0)
        m = v < 2
        mm = m if mm is None else (mm | m)
      fold_v[pl.ds(LANES, LANES)] = jnp.zeros((LANES,), jnp.int32)
      fold_v[pl.ds(0, LANES)] = jnp.where(mm, 1, 0)
      for off in (8, 4, 2, 1):
        fold_v[pl.ds(0, LANES)] = (
            fold_v[pl.ds(0, LANES)] | fold_v[pl.ds(off, LANES)])
      flags_v[pl.ds(c * LANES, LANES)] = fold_v[pl.ds(0, LANES)]
      return carry
    lax.fori_loop(0, nch, prep, 0)

    def start_gather(c, b):
      pltpu.make_async_copy(
          priori_hbm.at[aidx_v.at[c]], bufs.at[b], gsem.at[b]).start()

    def wait_gather(c, b):
      pltpu.make_async_copy(
          priori_hbm.at[aidx_v.at[c]], bufs.at[b], gsem.at[b]).wait()

    def out_copy(c, b):
      return pltpu.make_async_copy(
          bufs.at[b],
          out_hbm.at[pl.ds(base + c * CH, CH), pl.ds(0, DIM)], osem.at[b])

    def fixup(c, b):
      buf = bufs.at[b]
      flag = flags_v[pl.ds(c * LANES, LANES)][0]

      @pl.when(flag > 0)
      def _():
        def sub(s, carry):
          v = idx_v[pl.ds(c * CH + s * LANES, LANES)]
          for r in range(LANES):
            pv = v[r]

            @pl.when(pv < 2)
            def _():
              row = s * LANES + r
              for g in range(DIM // LANES):
                sl = pl.ds(g * LANES, LANES)
                k0 = kern_v[0, sl]
                k1 = kern_v[1, sl]
                buf[row, sl] = jnp.where(pv == 0, k0, k1)
          return carry
        lax.fori_loop(0, SUB, sub, 0)

    for b in range(NBUF):
      start_gather(b, b)

    def group(g, carry):
      for b in range(NBUF):
        c = g * NBUF + b
        wait_gather(c, b)
        fixup(c, b)
        out_copy(c, b).start()

        @pl.when(g < groups - 1)
        def _():
          out_copy(c, b).wait()
          start_gather(c + NBUF, b)
      return carry
    lax.fori_loop(0, groups, group, 0)

    for b in range(NBUF):
      c = (groups - 1) * NBUF + b
      out_copy(c, b).wait()

  return body(idx, kern, priori)


def kernel(inputs, kernel, priori):
  idx = inputs.astype(jnp.int32).reshape(-1)
  out = _sc_gather(idx, kernel, priori)
  return out[:, :DIM].reshape(inputs.shape + (DIM,))


# speedup vs baseline: 1.0292x; 1.0292x over previous
import functools

import jax
import jax.numpy as jnp
from jax import lax
from jax.experimental import pallas as pl
from jax.experimental.pallas import tpu as pltpu
from jax.experimental.pallas import tpu_sc as plsc

DIM = 64
NC = 2
NS = 16
NW = NC * NS
LANES = 16

CH = 128
NBUF = 10
SUB = CH // LANES


def _sc_gather(idx, kern, priori):
  B = idx.shape[0]
  rows_w = B // NW
  nch = rows_w // CH
  groups = nch // NBUF

  mesh = plsc.VectorSubcoreMesh(
      core_axis_name="c", subcore_axis_name="s",
      num_cores=NC, num_subcores=NS)

  @functools.partial(
      pl.kernel,
      out_type=jax.ShapeDtypeStruct((B, DIM), jnp.float32),
      mesh=mesh,
      compiler_params=pltpu.CompilerParams(use_tc_tiling_on_sc=False),
      scratch_types=[
          pltpu.VMEM((rows_w,), jnp.int32),
          pltpu.VMEM((nch, CH), jnp.int32),
          pltpu.VMEM((2, DIM), jnp.float32),
          pltpu.VMEM((nch * LANES,), jnp.int32),
          pltpu.VMEM((2 * LANES,), jnp.int32),
          pltpu.VMEM((NBUF, CH, DIM), jnp.float32),
          pltpu.SemaphoreType.DMA((NBUF,)),
          pltpu.SemaphoreType.DMA((NBUF,)),
      ],
  )
  def body(idx_hbm, kern_hbm, priori_hbm, out_hbm,
           idx_v, aidx_v, kern_v, flags_v, fold_v, bufs, gsem, osem):
    wid = lax.axis_index("s") * NC + lax.axis_index("c")
    base = wid * rows_w

    pltpu.sync_copy(idx_hbm.at[pl.ds(base, rows_w)], idx_v)
    pltpu.sync_copy(kern_hbm, kern_v)

    def prep(c, carry):
      mm = None
      for s in range(SUB):
        v = idx_v[pl.ds(c * CH + s * LANES, LANES)]
        aidx_v[c, pl.ds(s * LANES, LANES)] = jnp.maximum(v - 2, 0)
        m = v < 2
        mm = m if mm is None else (mm | m)
      fold_v[pl.ds(LANES, LANES)] = jnp.zeros((LANES,), jnp.int32)
      fold_v[pl.ds(0, LANES)] = jnp.where(mm, 1, 0)
      for off in (8, 4, 2, 1):
        fold_v[pl.ds(0, LANES)] = (
            fold_v[pl.ds(0, LANES)] | fold_v[pl.ds(off, LANES)])
      flags_v[pl.ds(c * LANES, LANES)] = fold_v[pl.ds(0, LANES)]
      return carry
    lax.fori_loop(0, nch, prep, 0)

    def start_gather(c, b):
      pltpu.make_async_copy(
          priori_hbm.at[aidx_v.at[c]], bufs.at[b], gsem.at[b]).start()

    def wait_gather(c, b):
      pltpu.make_async_copy(
          priori_hbm.at[aidx_v.at[c]], bufs.at[b], gsem.at[b]).wait()

    def out_copy(c, b):
      return pltpu.make_async_copy(
          bufs.at[b], out_hbm.at[pl.ds(base + c * CH, CH)], osem.at[b])

    def fixup(c, b):
      buf = bufs.at[b]
      flag = flags_v[pl.ds(c * LANES, LANES)][0]

      @pl.when(flag > 0)
      def _():
        def sub(s, carry):
          v = idx_v[pl.ds(c * CH + s * LANES, LANES)]
          for r in range(LANES):
            pv = v[r]

            @pl.when(pv < 2)
            def _():
              row = s * LANES + r
              for g in range(DIM // LANES):
                sl = pl.ds(g * LANES, LANES)
                k0 = kern_v[0, sl]
                k1 = kern_v[1, sl]
                buf[row, sl] = jnp.where(pv == 0, k0, k1)
          return carry
        lax.fori_loop(0, SUB, sub, 0)

    for b in range(NBUF):
      start_gather(b, b)

    def group(g, carry):
      for b in range(NBUF):
        c = g * NBUF + b
        wait_gather(c, b)
        fixup(c, b)
        out_copy(c, b).start()

        @pl.when(g < groups - 1)
        def _():
          out_copy(c, b).wait()
          start_gather(c + NBUF, b)
      return carry
    lax.fori_loop(0, groups, group, 0)

    for b in range(NBUF):
      c = (groups - 1) * NBUF + b
      out_copy(c, b).wait()

  return body(idx, kern, priori)


def kernel(inputs, kernel, priori):
  idx = inputs.astype(jnp.int32).reshape(-1)
  pflat = jax.lax.optimization_barrier(priori.reshape(-1))
  p2d = pflat.reshape(priori.shape)
  out = _sc_gather(idx, kernel, p2d)
  return out.reshape(inputs.shape + (DIM,))


# speedup vs baseline: 1.0924x; 1.0614x over previous
import functools

import jax
import jax.numpy as jnp
from jax import lax
from jax.experimental import pallas as pl
from jax.experimental.pallas import tpu as pltpu
from jax.experimental.pallas import tpu_sc as plsc

DIM = 64
NC = 2
NS = 16
NW = NC * NS
LANES = 16

CH = 128
NBUF = 5
PADW = 2 * DIM
SUB = CH // LANES


def _sc_gather(idx, kern, priori):
  B = idx.shape[0]
  rows_w = B // NW
  nch = rows_w // CH
  groups = nch // NBUF

  mesh = plsc.VectorSubcoreMesh(
      core_axis_name="c", subcore_axis_name="s",
      num_cores=NC, num_subcores=NS)

  @functools.partial(
      pl.kernel,
      out_type=jax.ShapeDtypeStruct((B, DIM), jnp.float32),
      mesh=mesh,
      compiler_params=pltpu.CompilerParams(use_tc_tiling_on_sc=False),
      scratch_types=[
          pltpu.VMEM((rows_w,), jnp.int32),
          pltpu.VMEM((nch, CH), jnp.int32),
          pltpu.VMEM((2, DIM), jnp.float32),
          pltpu.VMEM((nch * LANES,), jnp.int32),
          pltpu.VMEM((2 * LANES,), jnp.int32),
          pltpu.VMEM((NBUF, CH, PADW), jnp.float32),
          pltpu.SemaphoreType.DMA((NBUF,)),
          pltpu.SemaphoreType.DMA((NBUF,)),
      ],
  )
  def body(idx_hbm, kern_hbm, priori_hbm, out_hbm,
           idx_v, aidx_v, kern_v, flags_v, fold_v, bufs, gsem, osem):
    wid = lax.axis_index("s") * NC + lax.axis_index("c")
    base = wid * rows_w

    pltpu.sync_copy(idx_hbm.at[pl.ds(base, rows_w)], idx_v)
    pltpu.sync_copy(kern_hbm, kern_v)

    def prep(c, carry):
      mm = None
      for s in range(SUB):
        v = idx_v[pl.ds(c * CH + s * LANES, LANES)]
        aidx_v[c, pl.ds(s * LANES, LANES)] = jnp.maximum(v - 2, 0)
        m = v < 2
        mm = m if mm is None else (mm | m)
      fold_v[pl.ds(LANES, LANES)] = jnp.zeros((LANES,), jnp.int32)
      fold_v[pl.ds(0, LANES)] = jnp.where(mm, 1, 0)
      for off in (8, 4, 2, 1):
        fold_v[pl.ds(0, LANES)] = (
            fold_v[pl.ds(0, LANES)] | fold_v[pl.ds(off, LANES)])
      flags_v[pl.ds(c * LANES, LANES)] = fold_v[pl.ds(0, LANES)]
      return carry
    lax.fori_loop(0, nch, prep, 0)

    def start_gather(c, b):
      pltpu.make_async_copy(
          priori_hbm.at[aidx_v.at[c]], bufs.at[b], gsem.at[b]).start()

    def wait_gather(c, b):
      pltpu.make_async_copy(
          priori_hbm.at[aidx_v.at[c]], bufs.at[b], gsem.at[b]).wait()

    def out_copy(c, b):
      return pltpu.make_async_copy(
          bufs.at[b, :, pl.ds(0, DIM)],
          out_hbm.at[pl.ds(base + c * CH, CH)], osem.at[b])

    def fixup(c, b):
      buf = bufs.at[b]
      flag = flags_v[pl.ds(c * LANES, LANES)][0]

      @pl.when(flag > 0)
      def _():
        def sub(s, carry):
          v = idx_v[pl.ds(c * CH + s * LANES, LANES)]
          for r in range(LANES):
            pv = v[r]

            @pl.when(pv < 2)
            def _():
              row = s * LANES + r
              for g in range(DIM // LANES):
                sl = pl.ds(g * LANES, LANES)
                k0 = kern_v[0, sl]
                k1 = kern_v[1, sl]
                buf[row, sl] = jnp.where(pv == 0, k0, k1)
          return carry
        lax.fori_loop(0, SUB, sub, 0)

    for b in range(NBUF):
      start_gather(b, b)

    def group(g, carry):
      for b in range(NBUF):
        c = g * NBUF + b
        wait_gather(c, b)
        fixup(c, b)
        out_copy(c, b).start()

        @pl.when(g < groups - 1)
        def _():
          out_copy(c, b).wait()
          start_gather(c + NBUF, b)
      return carry
    lax.fori_loop(0, groups, group, 0)

    for b in range(NBUF):
      c = (groups - 1) * NBUF + b
      out_copy(c, b).wait()

  return body(idx, kern, priori)


def kernel(inputs, kernel, priori):
  idx = inputs.astype(jnp.int32).reshape(-1)
  p_pad = jnp.pad(priori, ((0, 0), (0, PADW - DIM)))
  out = _sc_gather(idx, kernel, p_pad)
  return out.reshape(inputs.shape + (DIM,))


# speedup vs baseline: 1.1214x; 1.0265x over previous
import functools

import jax
import jax.numpy as jnp
from jax import lax
from jax.experimental import pallas as pl
from jax.experimental.pallas import tpu as pltpu
from jax.experimental.pallas import tpu_sc as plsc

DIM = 64
NC = 2
NS = 16
NW = NC * NS
LANES = 16

CH = 128
NBUF = 10
PADW = 2 * DIM
SUB = CH // LANES


def _sc_gather(idx, kern, priori):
  B = idx.shape[0]
  rows_w = B // NW
  nch = rows_w // CH
  groups = nch // NBUF

  mesh = plsc.VectorSubcoreMesh(
      core_axis_name="c", subcore_axis_name="s",
      num_cores=NC, num_subcores=NS)

  @functools.partial(
      pl.kernel,
      out_type=jax.ShapeDtypeStruct((B, DIM), jnp.float32),
      mesh=mesh,
      compiler_params=pltpu.CompilerParams(use_tc_tiling_on_sc=False),
      scratch_types=[
          pltpu.VMEM((rows_w,), jnp.int32),
          pltpu.VMEM((nch, CH), jnp.int32),
          pltpu.VMEM((2, DIM), jnp.float32),
          pltpu.VMEM((nch * LANES,), jnp.int32),
          pltpu.VMEM((2 * LANES,), jnp.int32),
          pltpu.VMEM((NBUF, CH, DIM), jnp.float32),
          pltpu.SemaphoreType.DMA((NBUF,)),
          pltpu.SemaphoreType.DMA((NBUF,)),
      ],
  )
  def body(idx_hbm, kern_hbm, priori_hbm, out_hbm,
           idx_v, aidx_v, kern_v, flags_v, fold_v, bufs, gsem, osem):
    wid = lax.axis_index("s") * NC + lax.axis_index("c")
    base = wid * rows_w

    pltpu.sync_copy(idx_hbm.at[pl.ds(base, rows_w)], idx_v)
    pltpu.sync_copy(kern_hbm, kern_v)

    def prep(c, carry):
      mm = None
      for s in range(SUB):
        v = idx_v[pl.ds(c * CH + s * LANES, LANES)]
        aidx_v[c, pl.ds(s * LANES, LANES)] = jnp.maximum(2 * v - 4, 0)
        m = v < 2
        mm = m if mm is None else (mm | m)
      fold_v[pl.ds(LANES, LANES)] = jnp.zeros((LANES,), jnp.int32)
      fold_v[pl.ds(0, LANES)] = jnp.where(mm, 1, 0)
      for off in (8, 4, 2, 1):
        fold_v[pl.ds(0, LANES)] = (
            fold_v[pl.ds(0, LANES)] | fold_v[pl.ds(off, LANES)])
      flags_v[pl.ds(c * LANES, LANES)] = fold_v[pl.ds(0, LANES)]
      return carry
    lax.fori_loop(0, nch, prep, 0)

    def start_gather(c, b):
      pltpu.make_async_copy(
          priori_hbm.at[aidx_v.at[c]], bufs.at[b], gsem.at[b]).start()

    def wait_gather(c, b):
      pltpu.make_async_copy(
          priori_hbm.at[aidx_v.at[c]], bufs.at[b], gsem.at[b]).wait()

    def out_copy(c, b):
      return pltpu.make_async_copy(
          bufs.at[b], out_hbm.at[pl.ds(base + c * CH, CH)], osem.at[b])

    def fixup(c, b):
      buf = bufs.at[b]
      flag = flags_v[pl.ds(c * LANES, LANES)][0]

      @pl.when(flag > 0)
      def _():
        def sub(s, carry):
          v = idx_v[pl.ds(c * CH + s * LANES, LANES)]
          for r in range(LANES):
            pv = v[r]

            @pl.when(pv < 2)
            def _():
              row = s * LANES + r
              for g in range(DIM // LANES):
                sl = pl.ds(g * LANES, LANES)
                k0 = kern_v[0, sl]
                k1 = kern_v[1, sl]
                buf[row, sl] = jnp.where(pv == 0, k0, k1)
          return carry
        lax.fori_loop(0, SUB, sub, 0)

    for b in range(NBUF):
      start_gather(b, b)

    def group(g, carry):
      for b in range(NBUF):
        c = g * NBUF + b
        wait_gather(c, b)
        fixup(c, b)
        out_copy(c, b).start()

        @pl.when(g < groups - 1)
        def _():
          out_copy(c, b).wait()
          start_gather(c + NBUF, b)
      return carry
    lax.fori_loop(0, groups, group, 0)

    for b in range(NBUF):
      c = (groups - 1) * NBUF + b
      out_copy(c, b).wait()

  return body(idx, kern, priori)


def kernel(inputs, kernel, priori):
  idx = inputs.astype(jnp.int32).reshape(-1)
  p_pad = jnp.pad(priori, ((0, 0), (0, PADW - DIM)))
  out = _sc_gather(idx, kernel, p_pad.reshape(-1, DIM))
  return out.reshape(inputs.shape + (DIM,))
